# baseline (speedup 1.0000x reference)
import functools

import jax
import jax.numpy as jnp
from jax import lax
from jax.experimental import pallas as pl
from jax.experimental.pallas import tpu as pltpu
from jax.experimental.pallas import tpu_sc as plsc

N = 10000
K = 32
C_IN = 4
C = 128
C_HEAD = 256
N_RES = 4
DIFF_FACTOR = 40.0
EPS = 1e-5

NC = 2
NS = 16
LANES = 16
NW = NC * NS
RPW = 320
NP = NW * RPW

NE = N * K
EPW = NE // NW
EB = 80
SB = 25
ESUP = EB * SB
NSUP = EPW // ESUP

BATCH = 4
NB = RPW // BATCH

STAGE = 624
STAGE_REM = N - NS * STAGE


@functools.lru_cache(maxsize=1)
def _mesh():
    return plsc.VectorSubcoreMesh(
        core_axis_name="c", subcore_axis_name="s", num_cores=NC, num_subcores=NS
    )


_SC_PARAMS = pltpu.CompilerParams(use_tc_tiling_on_sc=False)


def _sc_edge_gather(table8, eidx):
    @functools.partial(
        pl.kernel,
        out_type=jax.ShapeDtypeStruct((NE, 8), jnp.float32),
        mesh=_mesh(),
        compiler_params=_SC_PARAMS,
        scratch_types=[
            pltpu.VMEM_SHARED((N, 8), jnp.float32),
            pltpu.VMEM((EPW,), jnp.int32),
            pltpu.VMEM((ESUP, 8), jnp.float32),
            pltpu.VMEM((ESUP, 8), jnp.float32),
            pltpu.SemaphoreType.DMA,
            pltpu.SemaphoreType.DMA,
        ],
    )
    def k(tab_hbm, idx_hbm, out_hbm, shared, idx_v, buf0, buf1, gsem, osem):
        cid = lax.axis_index("c")
        sid = lax.axis_index("s")
        wid = sid * NC + cid
        pltpu.sync_copy(
            tab_hbm.at[pl.ds(sid * STAGE, STAGE)],
            shared.at[pl.ds(sid * STAGE, STAGE)],
        )

        @pl.when(sid == NS - 1)
        def _():
            pltpu.sync_copy(
                tab_hbm.at[pl.ds(NS * STAGE, STAGE_REM)],
                shared.at[pl.ds(NS * STAGE, STAGE_REM)],
            )

        pltpu.sync_copy(idx_hbm.at[pl.ds(wid * EPW, EPW)], idx_v)
        plsc.subcore_barrier()

        bufs = (buf0, buf1)
        for s in range(NSUP):
            buf = bufs[s % 2]
            if s >= 2:
                pltpu.make_async_copy(buf, out_hbm.at[pl.ds(0, ESUP)], osem).wait()
            for u in range(SB):
                pltpu.async_copy(
                    shared.at[idx_v.at[pl.ds((s * SB + u) * EB, EB)]],
                    buf.at[pl.ds(u * EB, EB)],
                    gsem,
                )
            pltpu.make_async_copy(tab_hbm.at[pl.ds(0, ESUP)], buf, gsem).wait()
            pltpu.async_copy(
                buf, out_hbm.at[pl.ds(wid * EPW + s * ESUP, ESUP)], osem
            )
        for s in (NSUP - 2, NSUP - 1):
            pltpu.make_async_copy(bufs[s % 2], out_hbm.at[pl.ds(0, ESUP)], osem).wait()

    return k(table8, eidx)


def _sc_gather_max(table, idx1d):
    @functools.partial(
        pl.kernel,
        out_type=jax.ShapeDtypeStruct((NP, C), jnp.float32),
        mesh=_mesh(),
        compiler_params=_SC_PARAMS,
        scratch_types=[
            pltpu.VMEM_SHARED((N, C), jnp.float32),
            pltpu.VMEM((RPW * K,), jnp.int32),
            pltpu.VMEM((BATCH * K, C), jnp.float32),
            pltpu.VMEM((BATCH * K, C), jnp.float32),
            pltpu.VMEM((2 * BATCH, C), jnp.float32),
            pltpu.SemaphoreType.DMA,
            pltpu.SemaphoreType.DMA,
            pltpu.SemaphoreType.DMA,
        ],
    )
    def k(tab_hbm, idx_hbm, out_hbm, shared, idx_v, rows0, rows1, outb,
          gsem0, gsem1, osem):
        cid = lax.axis_index("c")
        sid = lax.axis_index("s")
        wid = sid * NC + cid
        pltpu.sync_copy(
            tab_hbm.at[pl.ds(sid * STAGE, STAGE)],
            shared.at[pl.ds(sid * STAGE, STAGE)],
        )

        @pl.when(sid == NS - 1)
        def _():
            pltpu.sync_copy(
                tab_hbm.at[pl.ds(NS * STAGE, STAGE_REM)],
                shared.at[pl.ds(NS * STAGE, STAGE_REM)],
            )

        pltpu.sync_copy(idx_hbm.at[pl.ds(wid * RPW * K, RPW * K)], idx_v)
        plsc.subcore_barrier()

        rows = (rows0, rows1)
        sems = (gsem0, gsem1)
        BK = BATCH * K

        def gdma(tb, u):
            src = shared if u == 0 else tab_hbm
            pltpu.async_copy(
                src.at[idx_v.at[pl.ds(tb * BK, BK)]], rows[u], sems[u]
            )

        gdma(0, 0)
        gdma(1, 1)

        @pl.loop(0, NB, step=2)
        def _(t0):
            @pl.when(t0 >= 2)
            def _():
                pltpu.make_async_copy(
                    outb, out_hbm.at[pl.ds(0, 2 * BATCH)], osem
                ).wait()

            for u in range(2):
                tb = t0 + u
                buf = rows[u]
                pltpu.make_async_copy(
                    tab_hbm.at[pl.ds(0, BK)], buf, sems[u]
                ).wait()

                def compute(b, u=u, buf=buf):
                    for c0 in range(0, C, LANES):
                        sl = pl.ds(c0, LANES)
                        acc = [buf[b * K + t, sl] for t in range(4)]
                        for j in range(4, K):
                            acc[j & 3] = jnp.maximum(acc[j & 3],
                                                     buf[b * K + j, sl])
                        outb[u * BATCH + b, sl] = jnp.maximum(
                            jnp.maximum(acc[0], acc[1]),
                            jnp.maximum(acc[2], acc[3]),
                        )

                pl.loop(0, BATCH)(compute)

                @pl.when(tb + 2 < NB)
                def _():
                    gdma(tb + 2, u)

            pltpu.async_copy(
                outb,
                out_hbm.at[pl.ds(wid * RPW + t0 * BATCH, 2 * BATCH)],
                osem,
            )

        pltpu.make_async_copy(outb, out_hbm.at[pl.ds(0, 2 * BATCH)], osem).wait()

    return k(table, idx1d)


_DOT = functools.partial(
    jax.lax.dot_general,
    dimension_numbers=(((1,), (0,)), ((), ())),
    preferred_element_type=jnp.float32,
    precision=jax.lax.Precision.DEFAULT,
)


def _bn(x, g, b):
    mu = jnp.mean(x, axis=0, keepdims=True)
    var = jnp.var(x, axis=0, keepdims=True)
    return (x - mu) / jnp.sqrt(var + EPS) * g + b


R1B = 400
G1 = N // R1B


def _tc_stage1(E, p, W_sa, W_pos):
    def body(e_ref, p_ref, wsa_ref, wpos_ref, o_ref):
        ev = e_ref[...]
        pe_in = ev[:, C_IN : C_IN + 3].reshape(R1B, K, 3)
        pc = (p_ref[...] * DIFF_FACTOR).reshape(R1B, 1, 3)
        rel = (pe_in - pc).reshape(R1B * K, 3)
        pe = jnp.maximum(_DOT(rel, wpos_ref[...]), 0.0)
        h0 = _DOT(ev[:, 0:C_IN], wsa_ref[...])
        nbr = (h0 + pe).reshape(R1B, K, C)
        o_ref[...] = jnp.max(nbr, axis=1)

    return pl.pallas_call(
        body,
        grid=(G1,),
        in_specs=[
            pl.BlockSpec((R1B * K, 8), lambda i: (i, 0)),
            pl.BlockSpec((R1B, 3), lambda i: (i, 0)),
            pl.BlockSpec((C_IN, C), lambda i: (0, 0)),
            pl.BlockSpec((3, C), lambda i: (0, 0)),
        ],
        out_specs=pl.BlockSpec((R1B, C), lambda i: (i, 0)),
        out_shape=jax.ShapeDtypeStruct((N, C), jnp.float32),
    )(E, p, W_sa, W_pos)


def _tc_sa_post(agg, g_sa, b_sa):
    def body(a_ref, g_ref, b_ref, o_ref):
        o_ref[...] = jnp.maximum(_bn(a_ref[...], g_ref[...], b_ref[...]), 0.0)

    return pl.pallas_call(
        body, out_shape=jax.ShapeDtypeStruct((N, C), jnp.float32)
    )(agg, g_sa, b_sa)


def _tc_round(agg, h, W1i, W2i, g1i, b1i):
    def body(a_ref, h_ref, w1_ref, w2_ref, g_ref, b_ref, o_ref):
        y = _bn(a_ref[...], g_ref[...], b_ref[...])
        y = jnp.maximum(_DOT(y, w1_ref[...]), 0.0)
        y = _DOT(y, w2_ref[...])
        o_ref[...] = h_ref[...] + y

    return pl.pallas_call(
        body, out_shape=jax.ShapeDtypeStruct((N, C), jnp.float32)
    )(agg, h, W1i, W2i, g1i, b1i)


def _tc_final(h, g_post, b_post, W_head):
    def body(h_ref, g_ref, b_ref, w_ref, o_ref):
        o_ref[...] = _DOT(_bn(h_ref[...], g_ref[...], b_ref[...]), w_ref[...])

    return pl.pallas_call(
        body, out_shape=jax.ShapeDtypeStruct((N, C_HEAD), jnp.float32)
    )(h, g_post, b_post, W_head)


def kernel(p, f, f_gs, group_idx, W_sa, W_pos, g_sa, b_sa, W1, W2, g1, b1,
           g_post, b_post, W_head):
    idx = group_idx.astype(jnp.int32)
    eidx = idx.reshape(NE)
    idx1d = jnp.concatenate(
        [idx, jnp.zeros((NP - N, K), jnp.int32)], axis=0
    ).reshape(NP * K)

    table8 = jnp.concatenate(
        [f, p * DIFF_FACTOR, jnp.zeros((N, 1), jnp.float32)], axis=1
    )
    E = _sc_edge_gather(table8, eidx)
    agg = _tc_stage1(E, p, W_sa, W_pos)
    h = _tc_sa_post(agg, g_sa.reshape(1, C), b_sa.reshape(1, C))
    for i in range(N_RES):
        agg = _sc_gather_max(h, idx1d)[:N]
        h = _tc_round(agg, h, W1[i], W2[i],
                      g1[i].reshape(1, C), b1[i].reshape(1, C))
    f_out = _tc_final(h, g_post.reshape(1, C), b_post.reshape(1, C), W_head)
    return (f_out, 0.0)


# speedup vs baseline: 1.9627x; 1.9627x over previous
import functools

import jax
import jax.numpy as jnp
from jax import lax
from jax.experimental import pallas as pl
from jax.experimental.pallas import tpu as pltpu
from jax.experimental.pallas import tpu_sc as plsc

N = 10000
K = 32
C_IN = 4
C = 128
C_HEAD = 256
N_RES = 4
DIFF_FACTOR = 40.0
EPS = 1e-5

NC = 2
NS = 16
LANES = 16
NW = NC * NS
RPW = 320
NP = NW * RPW

NE = N * K
EPW = NE // NW
EB = 80
SB = 25
ESUP = EB * SB
NSUP = EPW // ESUP

BATCH = 4
NB = RPW // BATCH

STAGE = 624
STAGE_REM = N - NS * STAGE


@functools.lru_cache(maxsize=1)
def _mesh():
    return plsc.VectorSubcoreMesh(
        core_axis_name="c", subcore_axis_name="s", num_cores=NC, num_subcores=NS
    )


_SC_PARAMS = pltpu.CompilerParams(use_tc_tiling_on_sc=False)


def _sc_edge_gather(table8, eidx):
    @functools.partial(
        pl.kernel,
        out_type=jax.ShapeDtypeStruct((NE, 8), jnp.float32),
        mesh=_mesh(),
        compiler_params=_SC_PARAMS,
        scratch_types=[
            pltpu.VMEM_SHARED((N, 8), jnp.float32),
            pltpu.VMEM((EPW,), jnp.int32),
            pltpu.VMEM((ESUP, 8), jnp.float32),
            pltpu.VMEM((ESUP, 8), jnp.float32),
            pltpu.SemaphoreType.DMA,
            pltpu.SemaphoreType.DMA,
        ],
    )
    def k(tab_hbm, idx_hbm, out_hbm, shared, idx_v, buf0, buf1, gsem, osem):
        cid = lax.axis_index("c")
        sid = lax.axis_index("s")
        wid = sid * NC + cid
        pltpu.sync_copy(
            tab_hbm.at[pl.ds(sid * STAGE, STAGE)],
            shared.at[pl.ds(sid * STAGE, STAGE)],
        )

        @pl.when(sid == NS - 1)
        def _():
            pltpu.sync_copy(
                tab_hbm.at[pl.ds(NS * STAGE, STAGE_REM)],
                shared.at[pl.ds(NS * STAGE, STAGE_REM)],
            )

        pltpu.sync_copy(idx_hbm.at[pl.ds(wid * EPW, EPW)], idx_v)
        plsc.subcore_barrier()

        bufs = (buf0, buf1)
        for s in range(NSUP):
            buf = bufs[s % 2]
            if s >= 2:
                pltpu.make_async_copy(buf, out_hbm.at[pl.ds(0, ESUP)], osem).wait()
            for u in range(SB):
                pltpu.async_copy(
                    shared.at[idx_v.at[pl.ds((s * SB + u) * EB, EB)]],
                    buf.at[pl.ds(u * EB, EB)],
                    gsem,
                )
            pltpu.make_async_copy(tab_hbm.at[pl.ds(0, ESUP)], buf, gsem).wait()
            pltpu.async_copy(
                buf, out_hbm.at[pl.ds(wid * EPW + s * ESUP, ESUP)], osem
            )
        for s in (NSUP - 2, NSUP - 1):
            pltpu.make_async_copy(bufs[s % 2], out_hbm.at[pl.ds(0, ESUP)], osem).wait()

    return k(table8, eidx)


def _sc_gather_max(table, idx1d):
    @functools.partial(
        pl.kernel,
        out_type=jax.ShapeDtypeStruct((NP, C), jnp.float32),
        mesh=_mesh(),
        compiler_params=_SC_PARAMS,
        scratch_types=[
            pltpu.VMEM_SHARED((N, C), jnp.float32),
            pltpu.VMEM((RPW * K,), jnp.int32),
            pltpu.VMEM((BATCH * K, C), jnp.float32),
            pltpu.VMEM((BATCH * K, C), jnp.float32),
            pltpu.VMEM((2 * BATCH, C), jnp.float32),
            pltpu.SemaphoreType.DMA,
            pltpu.SemaphoreType.DMA,
            pltpu.SemaphoreType.DMA,
        ],
    )
    def k(tab_hbm, idx_hbm, out_hbm, shared, idx_v, rows0, rows1, outb,
          gsem0, gsem1, osem):
        cid = lax.axis_index("c")
        sid = lax.axis_index("s")
        wid = sid * NC + cid
        pltpu.sync_copy(
            tab_hbm.at[pl.ds(sid * STAGE, STAGE)],
            shared.at[pl.ds(sid * STAGE, STAGE)],
        )

        @pl.when(sid == NS - 1)
        def _():
            pltpu.sync_copy(
                tab_hbm.at[pl.ds(NS * STAGE, STAGE_REM)],
                shared.at[pl.ds(NS * STAGE, STAGE_REM)],
            )

        pltpu.sync_copy(idx_hbm.at[pl.ds(wid * RPW * K, RPW * K)], idx_v)
        plsc.subcore_barrier()

        rows = (rows0, rows1)
        sems = (gsem0, gsem1)
        BK = BATCH * K

        def gdma(tb, u):
            pltpu.async_copy(
                shared.at[idx_v.at[pl.ds(tb * BK, BK)]], rows[u], sems[u]
            )

        gdma(0, 0)
        gdma(1, 1)

        @pl.loop(0, NB, step=2)
        def _(t0):
            @pl.when(t0 >= 2)
            def _():
                pltpu.make_async_copy(
                    outb, out_hbm.at[pl.ds(0, 2 * BATCH)], osem
                ).wait()

            for u in range(2):
                tb = t0 + u
                buf = rows[u]
                pltpu.make_async_copy(
                    tab_hbm.at[pl.ds(0, BK)], buf, sems[u]
                ).wait()

                def compute(b, u=u, buf=buf):
                    for c0 in range(0, C, LANES):
                        sl = pl.ds(c0, LANES)
                        acc = [buf[b * K + t, sl] for t in range(4)]
                        for j in range(4, K):
                            acc[j & 3] = jnp.maximum(acc[j & 3],
                                                     buf[b * K + j, sl])
                        outb[u * BATCH + b, sl] = jnp.maximum(
                            jnp.maximum(acc[0], acc[1]),
                            jnp.maximum(acc[2], acc[3]),
                        )

                pl.loop(0, BATCH)(compute)

                @pl.when(tb + 2 < NB)
                def _():
                    gdma(tb + 2, u)

            pltpu.async_copy(
                outb,
                out_hbm.at[pl.ds(wid * RPW + t0 * BATCH, 2 * BATCH)],
                osem,
            )

        pltpu.make_async_copy(outb, out_hbm.at[pl.ds(0, 2 * BATCH)], osem).wait()

    return k(table, idx1d)


_DOT = functools.partial(
    jax.lax.dot_general,
    dimension_numbers=(((1,), (0,)), ((), ())),
    preferred_element_type=jnp.float32,
    precision=jax.lax.Precision.DEFAULT,
)


def _bn(x, g, b):
    mu = jnp.mean(x, axis=0, keepdims=True)
    var = jnp.var(x, axis=0, keepdims=True)
    return (x - mu) / jnp.sqrt(var + EPS) * g + b


R1B = 400
G1 = N // R1B


def _tc_stage1(E, p, W_sa, W_pos):
    def body(e_ref, p_ref, wsa_ref, wpos_ref, o_ref):
        ev = e_ref[...]
        pe_in = ev[:, C_IN : C_IN + 3].reshape(R1B, K, 3)
        pc = (p_ref[...] * DIFF_FACTOR).reshape(R1B, 1, 3)
        rel = (pe_in - pc).reshape(R1B * K, 3)
        pe = jnp.maximum(_DOT(rel, wpos_ref[...]), 0.0)
        h0 = _DOT(ev[:, 0:C_IN], wsa_ref[...])
        nbr = (h0 + pe).reshape(R1B, K, C)
        o_ref[...] = jnp.max(nbr, axis=1)

    return pl.pallas_call(
        body,
        grid=(G1,),
        in_specs=[
            pl.BlockSpec((R1B * K, 8), lambda i: (i, 0)),
            pl.BlockSpec((R1B, 3), lambda i: (i, 0)),
            pl.BlockSpec((C_IN, C), lambda i: (0, 0)),
            pl.BlockSpec((3, C), lambda i: (0, 0)),
        ],
        out_specs=pl.BlockSpec((R1B, C), lambda i: (i, 0)),
        out_shape=jax.ShapeDtypeStruct((N, C), jnp.float32),
    )(E, p, W_sa, W_pos)


def _tc_sa_post(agg, g_sa, b_sa):
    def body(a_ref, g_ref, b_ref, o_ref):
        o_ref[...] = jnp.maximum(_bn(a_ref[...], g_ref[...], b_ref[...]), 0.0)

    return pl.pallas_call(
        body, out_shape=jax.ShapeDtypeStruct((N, C), jnp.float32)
    )(agg, g_sa, b_sa)


def _tc_round(agg, h, W1i, W2i, g1i, b1i):
    def body(a_ref, h_ref, w1_ref, w2_ref, g_ref, b_ref, o_ref):
        y = _bn(a_ref[...], g_ref[...], b_ref[...])
        y = jnp.maximum(_DOT(y, w1_ref[...]), 0.0)
        y = _DOT(y, w2_ref[...])
        o_ref[...] = h_ref[...] + y

    return pl.pallas_call(
        body, out_shape=jax.ShapeDtypeStruct((N, C), jnp.float32)
    )(agg, h, W1i, W2i, g1i, b1i)


def _tc_final(h, g_post, b_post, W_head):
    def body(h_ref, g_ref, b_ref, w_ref, o_ref):
        o_ref[...] = _DOT(_bn(h_ref[...], g_ref[...], b_ref[...]), w_ref[...])

    return pl.pallas_call(
        body, out_shape=jax.ShapeDtypeStruct((N, C_HEAD), jnp.float32)
    )(h, g_post, b_post, W_head)


def kernel(p, f, f_gs, group_idx, W_sa, W_pos, g_sa, b_sa, W1, W2, g1, b1,
           g_post, b_post, W_head):
    idx = group_idx.astype(jnp.int32)
    eidx = idx.reshape(NE)
    idx1d = jnp.concatenate(
        [idx, jnp.zeros((NP - N, K), jnp.int32)], axis=0
    ).reshape(NP * K)

    table8 = jnp.concatenate(
        [f, p * DIFF_FACTOR, jnp.zeros((N, 1), jnp.float32)], axis=1
    )
    E = _sc_edge_gather(table8, eidx)
    agg = _tc_stage1(E, p, W_sa, W_pos)
    h = _tc_sa_post(agg, g_sa.reshape(1, C), b_sa.reshape(1, C))
    for i in range(N_RES):
        agg = _sc_gather_max(h, idx1d)[:N]
        h = _tc_round(agg, h, W1[i], W2[i],
                      g1[i].reshape(1, C), b1[i].reshape(1, C))
    f_out = _tc_final(h, g_post.reshape(1, C), b_post.reshape(1, C), W_head)
    return (f_out, 0.0)


# speedup vs baseline: 2.0116x; 1.0249x over previous
import functools

import jax
import jax.numpy as jnp
from jax import lax
from jax.experimental import pallas as pl
from jax.experimental.pallas import tpu as pltpu
from jax.experimental.pallas import tpu_sc as plsc

N = 10000
K = 32
C_IN = 4
C = 128
C_HEAD = 256
N_RES = 4
DIFF_FACTOR = 40.0
EPS = 1e-5

NC = 2
NS = 16
LANES = 16
NW = NC * NS
RPW = 320
NP = NW * RPW

NE = N * K
EPW = NE // NW
EB = 80
SB = 25
ESUP = EB * SB
NSUP = EPW // ESUP

BATCH = 4
NB = RPW // BATCH

STAGE = 624
STAGE_REM = N - NS * STAGE


@functools.lru_cache(maxsize=1)
def _mesh():
    return plsc.VectorSubcoreMesh(
        core_axis_name="c", subcore_axis_name="s", num_cores=NC, num_subcores=NS
    )


_SC_PARAMS = pltpu.CompilerParams(use_tc_tiling_on_sc=False)


def _sc_edge_gather(table8, eidx):
    @functools.partial(
        pl.kernel,
        out_type=jax.ShapeDtypeStruct((NE, 8), jnp.float32),
        mesh=_mesh(),
        compiler_params=_SC_PARAMS,
        scratch_types=[
            pltpu.VMEM_SHARED((N, 8), jnp.float32),
            pltpu.VMEM((EPW,), jnp.int32),
            pltpu.VMEM((ESUP, 8), jnp.float32),
            pltpu.VMEM((ESUP, 8), jnp.float32),
            pltpu.SemaphoreType.DMA,
            pltpu.SemaphoreType.DMA,
        ],
    )
    def k(tab_hbm, idx_hbm, out_hbm, shared, idx_v, buf0, buf1, gsem, osem):
        cid = lax.axis_index("c")
        sid = lax.axis_index("s")
        wid = sid * NC + cid
        pltpu.sync_copy(
            tab_hbm.at[pl.ds(sid * STAGE, STAGE)],
            shared.at[pl.ds(sid * STAGE, STAGE)],
        )

        @pl.when(sid == NS - 1)
        def _():
            pltpu.sync_copy(
                tab_hbm.at[pl.ds(NS * STAGE, STAGE_REM)],
                shared.at[pl.ds(NS * STAGE, STAGE_REM)],
            )

        pltpu.sync_copy(idx_hbm.at[pl.ds(wid * EPW, EPW)], idx_v)
        plsc.subcore_barrier()

        bufs = (buf0, buf1)
        for s in range(NSUP):
            buf = bufs[s % 2]
            if s >= 2:
                pltpu.make_async_copy(buf, out_hbm.at[pl.ds(0, ESUP)], osem).wait()
            for u in range(SB):
                pltpu.async_copy(
                    shared.at[idx_v.at[pl.ds((s * SB + u) * EB, EB)]],
                    buf.at[pl.ds(u * EB, EB)],
                    gsem,
                )
            pltpu.make_async_copy(tab_hbm.at[pl.ds(0, ESUP)], buf, gsem).wait()
            pltpu.async_copy(
                buf, out_hbm.at[pl.ds(wid * EPW + s * ESUP, ESUP)], osem
            )
        for s in (NSUP - 2, NSUP - 1):
            pltpu.make_async_copy(bufs[s % 2], out_hbm.at[pl.ds(0, ESUP)], osem).wait()

    return k(table8, eidx)


def _sc_gather_max(table, idx1d):
    @functools.partial(
        pl.kernel,
        out_type=jax.ShapeDtypeStruct((NP, C), jnp.float32),
        mesh=_mesh(),
        compiler_params=_SC_PARAMS,
        scratch_types=[
            pltpu.VMEM_SHARED((N, C), jnp.float32),
            pltpu.VMEM((RPW * K,), jnp.int32),
            pltpu.VMEM((BATCH * K, C), jnp.float32),
            pltpu.VMEM((BATCH * K, C), jnp.float32),
            pltpu.VMEM((2 * BATCH, C), jnp.float32),
            pltpu.SemaphoreType.DMA,
            pltpu.SemaphoreType.DMA,
        ],
    )
    def k(tab_hbm, idx_hbm, out_hbm, shared, idx_v, rows0, rows1, outb,
          gsem, osem):
        cid = lax.axis_index("c")
        sid = lax.axis_index("s")
        wid = sid * NC + cid
        pltpu.sync_copy(
            tab_hbm.at[pl.ds(sid * STAGE, STAGE)],
            shared.at[pl.ds(sid * STAGE, STAGE)],
        )

        @pl.when(sid == NS - 1)
        def _():
            pltpu.sync_copy(
                tab_hbm.at[pl.ds(NS * STAGE, STAGE_REM)],
                shared.at[pl.ds(NS * STAGE, STAGE_REM)],
            )

        pltpu.sync_copy(idx_hbm.at[pl.ds(wid * RPW * K, RPW * K)], idx_v)
        plsc.subcore_barrier()

        rows = (rows0, rows1)
        BK = BATCH * K

        def gdma(tb, u):
            pltpu.async_copy(
                shared.at[idx_v.at[pl.ds(tb * BK, BK)]], rows[u], gsem
            )

        gdma(0, 0)
        gdma(1, 1)

        @pl.loop(0, NB, step=2)
        def _(t0):
            @pl.when(t0 >= 2)
            def _():
                pltpu.make_async_copy(
                    outb, out_hbm.at[pl.ds(0, 2 * BATCH)], osem
                ).wait()

            for u in range(2):
                tb = t0 + u
                buf = rows[u]
                pltpu.make_async_copy(
                    tab_hbm.at[pl.ds(0, BK)], buf, gsem
                ).wait()

                def compute(b, u=u, buf=buf):
                    for c0 in range(0, C, LANES):
                        sl = pl.ds(c0, LANES)
                        acc = [buf[b * K + t, sl] for t in range(4)]
                        for j in range(4, K):
                            acc[j & 3] = jnp.maximum(acc[j & 3],
                                                     buf[b * K + j, sl])
                        outb[u * BATCH + b, sl] = jnp.maximum(
                            jnp.maximum(acc[0], acc[1]),
                            jnp.maximum(acc[2], acc[3]),
                        )

                pl.loop(0, BATCH)(compute)

                @pl.when(tb + 2 < NB)
                def _():
                    gdma(tb + 2, u)

            pltpu.async_copy(
                outb,
                out_hbm.at[pl.ds(wid * RPW + t0 * BATCH, 2 * BATCH)],
                osem,
            )

        pltpu.make_async_copy(outb, out_hbm.at[pl.ds(0, 2 * BATCH)], osem).wait()

    return k(table, idx1d)


_DOT = functools.partial(
    jax.lax.dot_general,
    dimension_numbers=(((1,), (0,)), ((), ())),
    preferred_element_type=jnp.float32,
    precision=jax.lax.Precision.DEFAULT,
)


def _bn(x, g, b):
    mu = jnp.mean(x, axis=0, keepdims=True)
    var = jnp.var(x, axis=0, keepdims=True)
    return (x - mu) / jnp.sqrt(var + EPS) * g + b


R1B = 400
G1 = N // R1B


def _tc_stage1(E, p, W_sa, W_pos):
    def body(e_ref, p_ref, wsa_ref, wpos_ref, o_ref):
        ev = e_ref[...]
        pe_in = ev[:, C_IN : C_IN + 3].reshape(R1B, K, 3)
        pc = (p_ref[...] * DIFF_FACTOR).reshape(R1B, 1, 3)
        rel = (pe_in - pc).reshape(R1B * K, 3)
        pe = jnp.maximum(_DOT(rel, wpos_ref[...]), 0.0)
        h0 = _DOT(ev[:, 0:C_IN], wsa_ref[...])
        nbr = (h0 + pe).reshape(R1B, K, C)
        o_ref[...] = jnp.max(nbr, axis=1)

    return pl.pallas_call(
        body,
        grid=(G1,),
        in_specs=[
            pl.BlockSpec((R1B * K, 8), lambda i: (i, 0)),
            pl.BlockSpec((R1B, 3), lambda i: (i, 0)),
            pl.BlockSpec((C_IN, C), lambda i: (0, 0)),
            pl.BlockSpec((3, C), lambda i: (0, 0)),
        ],
        out_specs=pl.BlockSpec((R1B, C), lambda i: (i, 0)),
        out_shape=jax.ShapeDtypeStruct((N, C), jnp.float32),
    )(E, p, W_sa, W_pos)


def _tc_sa_post(agg, g_sa, b_sa):
    def body(a_ref, g_ref, b_ref, o_ref):
        o_ref[...] = jnp.maximum(_bn(a_ref[...], g_ref[...], b_ref[...]), 0.0)

    return pl.pallas_call(
        body, out_shape=jax.ShapeDtypeStruct((N, C), jnp.float32)
    )(agg, g_sa, b_sa)


def _tc_round(agg, h, W1i, W2i, g1i, b1i):
    def body(a_ref, h_ref, w1_ref, w2_ref, g_ref, b_ref, o_ref):
        y = _bn(a_ref[...], g_ref[...], b_ref[...])
        y = jnp.maximum(_DOT(y, w1_ref[...]), 0.0)
        y = _DOT(y, w2_ref[...])
        o_ref[...] = h_ref[...] + y

    return pl.pallas_call(
        body, out_shape=jax.ShapeDtypeStruct((N, C), jnp.float32)
    )(agg, h, W1i, W2i, g1i, b1i)


def _tc_final(h, g_post, b_post, W_head):
    def body(h_ref, g_ref, b_ref, w_ref, o_ref):
        o_ref[...] = _DOT(_bn(h_ref[...], g_ref[...], b_ref[...]), w_ref[...])

    return pl.pallas_call(
        body, out_shape=jax.ShapeDtypeStruct((N, C_HEAD), jnp.float32)
    )(h, g_post, b_post, W_head)


def kernel(p, f, f_gs, group_idx, W_sa, W_pos, g_sa, b_sa, W1, W2, g1, b1,
           g_post, b_post, W_head):
    idx = group_idx.astype(jnp.int32)
    eidx = idx.reshape(NE)
    idx1d = jnp.concatenate(
        [idx, jnp.zeros((NP - N, K), jnp.int32)], axis=0
    ).reshape(NP * K)

    table8 = jnp.concatenate(
        [f, p * DIFF_FACTOR, jnp.zeros((N, 1), jnp.float32)], axis=1
    )
    E = _sc_edge_gather(table8, eidx)
    agg = _tc_stage1(E, p, W_sa, W_pos)
    h = _tc_sa_post(agg, g_sa.reshape(1, C), b_sa.reshape(1, C))
    for i in range(N_RES):
        agg = _sc_gather_max(h, idx1d)[:N]
        h = _tc_round(agg, h, W1[i], W2[i],
                      g1[i].reshape(1, C), b1[i].reshape(1, C))
    f_out = _tc_final(h, g_post.reshape(1, C), b_post.reshape(1, C), W_head)
    return (f_out, 0.0)
